# baseline (speedup 1.0000x reference)
import functools

import jax
import jax.numpy as jnp
from jax import lax
from jax.experimental import pallas as pl
from jax.experimental.pallas import tpu as pltpu
from jax.experimental.pallas import tpu_sc as plsc

N = 10000
E = 320000
D0 = 128
H1 = 256
D2 = 64

NC = 2
NS = 16
NW = NC * NS
EPW = E // NW
K = 80
C = 126
EPP = C * K
NP = 10240
RPS = NP // NS
ZR = 32
R = 1000

_DEG_W = 16


def _vector_mesh():
    return plsc.VectorSubcoreMesh(core_axis_name="c", subcore_axis_name="s")


def _zero_fill(zeros_v, acc_sh, base, width):
    for i in range(ZR):
        for j in range(width // 16):
            zeros_v[i, pl.ds(j * 16, 16)] = jnp.zeros((16,), jnp.float32)
    for kk in range(RPS // ZR):
        pltpu.sync_copy(zeros_v, acc_sh.at[pl.ds(base + kk * ZR, ZR)])


def _degree_partials(dst_r):
    @functools.partial(
        pl.kernel,
        out_type=jax.ShapeDtypeStruct((NC * NP, _DEG_W), jnp.float32),
        mesh=_vector_mesh(),
        scratch_types=[
            pltpu.VMEM((C, K), jnp.int32),
            pltpu.VMEM((K, _DEG_W), jnp.float32),
            pltpu.VMEM((ZR, _DEG_W), jnp.float32),
            pltpu.VMEM_SHARED((NP, _DEG_W), jnp.float32),
            pltpu.SemaphoreType.DMA,
        ],
    )
    def deg_kernel(dst_hbm, out_hbm, dst_v, ones_v, zeros_v, acc_sh, sem):
        c = lax.axis_index("c")
        s = lax.axis_index("s")
        wid = c * NS + s
        base = s * RPS
        for i in range(K):
            ones_v[i, :] = jnp.full((_DEG_W,), 1.0, jnp.float32)
        _zero_fill(zeros_v, acc_sh, base, _DEG_W)
        pltpu.sync_copy(dst_hbm.at[wid], dst_v)
        plsc.subcore_barrier()

        @pl.loop(0, C)
        def _(i):
            pltpu.async_copy(ones_v, acc_sh.at[dst_v.at[i]], sem, add=True)

        @pl.loop(0, C)
        def _(i):
            pltpu.make_async_copy(ones_v, acc_sh.at[dst_v.at[i]], sem).wait()

        plsc.subcore_barrier()
        pltpu.sync_copy(acc_sh.at[pl.ds(base, RPS)],
                        out_hbm.at[pl.ds(c * NP + base, RPS)])

    return deg_kernel(dst_r)


def _propagate_partials(g, src_r, dst_r, d):
    @functools.partial(
        pl.kernel,
        out_type=jax.ShapeDtypeStruct((NC * NP, d), jnp.float32),
        mesh=_vector_mesh(),
        scratch_types=[
            pltpu.VMEM((C, K), jnp.int32),
            pltpu.VMEM((C, K), jnp.int32),
            pltpu.VMEM((K, d), jnp.float32),
            pltpu.VMEM((ZR, d), jnp.float32),
            pltpu.VMEM_SHARED((NP, d), jnp.float32),
            pltpu.SemaphoreType.DMA,
        ],
    )
    def prop_kernel(g_hbm, src_hbm, dst_hbm, out_hbm,
                    src_v, dst_v, rows_v, zeros_v, acc_sh, sem):
        c = lax.axis_index("c")
        s = lax.axis_index("s")
        wid = c * NS + s
        base = s * RPS
        _zero_fill(zeros_v, acc_sh, base, d)
        pltpu.sync_copy(src_hbm.at[wid], src_v)
        pltpu.sync_copy(dst_hbm.at[wid], dst_v)
        plsc.subcore_barrier()

        @pl.loop(0, C)
        def _(i):
            pltpu.async_copy(g_hbm.at[src_v.at[i]], rows_v, sem).wait()
            pltpu.sync_copy(rows_v, acc_sh.at[dst_v.at[i]], add=True)

        plsc.subcore_barrier()
        pltpu.sync_copy(acc_sh.at[pl.ds(base, RPS)],
                        out_hbm.at[pl.ds(c * NP + base, RPS)])

    return prop_kernel(g, src_r, dst_r)


def _dinv_from(dp_ref):
    deg = dp_ref[0, :, 0:1] + dp_ref[1, :, 0:1] + 1.0
    return lax.rsqrt(deg)


def _tc_scale(x, degp):
    def body(x_ref, dp_ref, o_ref):
        o_ref[...] = x_ref[...] * _dinv_from(dp_ref)

    return pl.pallas_call(
        body,
        grid=(N // R,),
        in_specs=[
            pl.BlockSpec((R, D0), lambda r: (r, 0)),
            pl.BlockSpec((NC, R, _DEG_W), lambda r: (0, r, 0)),
        ],
        out_specs=pl.BlockSpec((R, D0), lambda r: (r, 0)),
        out_shape=jax.ShapeDtypeStruct((N, D0), jnp.float32),
    )(x, degp)


def _mm(a, b):
    return lax.dot_general(a, b, (((1,), (0,)), ((), ())),
                           precision=lax.Precision.HIGHEST,
                           preferred_element_type=jnp.float32)


def _tc_dense(s0p, g0, degp, W1, W2):
    def body(sp_ref, g_ref, dp_ref, w1_ref, w2_ref, o_ref):
        dinv = _dinv_from(dp_ref)
        t = (sp_ref[0] + sp_ref[1] + g_ref[...]) * dinv
        h1 = jnp.maximum(_mm(t, w1_ref[...]), 0.0)
        o_ref[...] = _mm(h1, w2_ref[...]) * dinv

    return pl.pallas_call(
        body,
        grid=(N // R,),
        in_specs=[
            pl.BlockSpec((NC, R, D0), lambda r: (0, r, 0)),
            pl.BlockSpec((R, D0), lambda r: (r, 0)),
            pl.BlockSpec((NC, R, _DEG_W), lambda r: (0, r, 0)),
            pl.BlockSpec((D0, H1), lambda r: (0, 0)),
            pl.BlockSpec((H1, D0), lambda r: (0, 0)),
        ],
        out_specs=pl.BlockSpec((R, D0), lambda r: (r, 0)),
        out_shape=jax.ShapeDtypeStruct((N, D0), jnp.float32),
    )(s0p, g0, degp, W1, W2)


def _tc_final(s2p, g2, degp):
    def body(sp_ref, g_ref, dp_ref, o_ref):
        acc = sp_ref[0, :, 0:D2] + sp_ref[1, :, 0:D2] + g_ref[:, 0:D2]
        o_ref[...] = acc * _dinv_from(dp_ref)

    return pl.pallas_call(
        body,
        grid=(N // R,),
        in_specs=[
            pl.BlockSpec((NC, R, D0), lambda r: (0, r, 0)),
            pl.BlockSpec((R, D0), lambda r: (r, 0)),
            pl.BlockSpec((NC, R, _DEG_W), lambda r: (0, r, 0)),
        ],
        out_specs=pl.BlockSpec((R, D2), lambda r: (r, 0)),
        out_shape=jax.ShapeDtypeStruct((N, D2), jnp.float32),
    )(s2p, g2, degp)


def kernel(x, edge_index, W1, W2):
    pad = EPP - EPW
    src = edge_index[0].astype(jnp.int32).reshape(NW, EPW)
    dst = edge_index[1].astype(jnp.int32).reshape(NW, EPW)
    src = jnp.concatenate(
        [src, jnp.zeros((NW, pad), jnp.int32)], axis=1).reshape(NW, C, K)
    dst = jnp.concatenate(
        [dst, jnp.full((NW, pad), NP - 1, jnp.int32)],
        axis=1).reshape(NW, C, K)

    W2p = jnp.concatenate(
        [W2, jnp.zeros((H1, D0 - D2), jnp.float32)], axis=1)

    degp = _degree_partials(dst).reshape(NC, NP, _DEG_W)
    g0 = _tc_scale(x, degp)
    s0p = _propagate_partials(g0, src, dst, D0).reshape(NC, NP, D0)
    g2 = _tc_dense(s0p, g0, degp, W1, W2p)
    s2p = _propagate_partials(g2, src, dst, D0).reshape(NC, NP, D0)
    return _tc_final(s2p, g2, degp)


# speedup vs baseline: 1.3194x; 1.3194x over previous
import functools

import jax
import jax.numpy as jnp
from jax import lax
from jax.experimental import pallas as pl
from jax.experimental.pallas import tpu as pltpu
from jax.experimental.pallas import tpu_sc as plsc

N = 10000
E = 320000
D0 = 128
H1 = 256
D2 = 64

NC = 2
NS = 16
NW = NC * NS
EPW = E // NW
K = 80
C = EPW // K
NP = 10240
RPS = NP // NS
ZR = 32
R = 1000

_DEG_W = 16


def _vector_mesh():
    return plsc.VectorSubcoreMesh(core_axis_name="c", subcore_axis_name="s")


def _zero_fill(zeros_v, acc_sh, base, width):
    for i in range(ZR):
        for j in range(width // 16):
            zeros_v[i, pl.ds(j * 16, 16)] = jnp.zeros((16,), jnp.float32)
    for kk in range(RPS // ZR):
        pltpu.sync_copy(zeros_v, acc_sh.at[pl.ds(base + kk * ZR, ZR)])


def _degree_partials(dst_r):
    @functools.partial(
        pl.kernel,
        out_type=jax.ShapeDtypeStruct((NC * NP, _DEG_W), jnp.float32),
        mesh=_vector_mesh(),
        scratch_types=[
            pltpu.VMEM((C, K), jnp.int32),
            pltpu.VMEM((K, _DEG_W), jnp.float32),
            pltpu.VMEM((ZR, _DEG_W), jnp.float32),
            pltpu.VMEM_SHARED((NP, _DEG_W), jnp.float32),
        ],
    )
    def deg_kernel(dst_hbm, out_hbm, dst_v, ones_v, zeros_v, acc_sh):
        c = lax.axis_index("c")
        s = lax.axis_index("s")
        wid = c * NS + s
        base = s * RPS
        for i in range(K):
            ones_v[i, :] = jnp.full((_DEG_W,), 1.0, jnp.float32)
        _zero_fill(zeros_v, acc_sh, base, _DEG_W)
        pltpu.sync_copy(dst_hbm.at[wid], dst_v)
        plsc.subcore_barrier()

        @pl.loop(0, C)
        def _(i):
            pltpu.sync_copy(ones_v, acc_sh.at[dst_v.at[i]], add=True)

        plsc.subcore_barrier()
        pltpu.sync_copy(acc_sh.at[pl.ds(base, RPS)],
                        out_hbm.at[pl.ds(c * NP + base, RPS)])

    return deg_kernel(dst_r)


def _propagate_partials(g, src_r, dst_r, d):
    @functools.partial(
        pl.kernel,
        out_type=jax.ShapeDtypeStruct((NC * NP, d), jnp.float32),
        mesh=_vector_mesh(),
        scratch_types=[
            pltpu.VMEM((C, K), jnp.int32),
            pltpu.VMEM((C, K), jnp.int32),
            pltpu.VMEM((K, d), jnp.float32),
            pltpu.VMEM((ZR, d), jnp.float32),
            pltpu.VMEM_SHARED((NP, d), jnp.float32),
            pltpu.SemaphoreType.DMA,
        ],
    )
    def prop_kernel(g_hbm, src_hbm, dst_hbm, out_hbm,
                    src_v, dst_v, rows_v, zeros_v, acc_sh, sem):
        c = lax.axis_index("c")
        s = lax.axis_index("s")
        wid = c * NS + s
        base = s * RPS
        _zero_fill(zeros_v, acc_sh, base, d)
        pltpu.sync_copy(src_hbm.at[wid], src_v)
        pltpu.sync_copy(dst_hbm.at[wid], dst_v)
        plsc.subcore_barrier()

        @pl.loop(0, C)
        def _(i):
            pltpu.async_copy(g_hbm.at[src_v.at[i]], rows_v, sem).wait()
            pltpu.sync_copy(rows_v, acc_sh.at[dst_v.at[i]], add=True)

        plsc.subcore_barrier()
        pltpu.sync_copy(acc_sh.at[pl.ds(base, RPS)],
                        out_hbm.at[pl.ds(c * NP + base, RPS)])

    return prop_kernel(g, src_r, dst_r)


def _dinv_from(dp_ref):
    deg = dp_ref[0, :, 0:1] + dp_ref[1, :, 0:1] + 1.0
    return lax.rsqrt(deg)


def _tc_scale(x, degp):
    def body(x_ref, dp_ref, o_ref):
        o_ref[...] = x_ref[...] * _dinv_from(dp_ref)

    return pl.pallas_call(
        body,
        grid=(N // R,),
        in_specs=[
            pl.BlockSpec((R, D0), lambda r: (r, 0)),
            pl.BlockSpec((NC, R, _DEG_W), lambda r: (0, r, 0)),
        ],
        out_specs=pl.BlockSpec((R, D0), lambda r: (r, 0)),
        out_shape=jax.ShapeDtypeStruct((N, D0), jnp.float32),
    )(x, degp)


def _mm(a, b):
    return lax.dot_general(a, b, (((1,), (0,)), ((), ())),
                           precision=lax.Precision.HIGHEST,
                           preferred_element_type=jnp.float32)


def _tc_dense(s0p, g0, degp, W1, W2):
    def body(sp_ref, g_ref, dp_ref, w1_ref, w2_ref, o_ref):
        dinv = _dinv_from(dp_ref)
        t = (sp_ref[0] + sp_ref[1] + g_ref[...]) * dinv
        h1 = jnp.maximum(_mm(t, w1_ref[...]), 0.0)
        o_ref[...] = _mm(h1, w2_ref[...]) * dinv

    return pl.pallas_call(
        body,
        grid=(N // R,),
        in_specs=[
            pl.BlockSpec((NC, R, D0), lambda r: (0, r, 0)),
            pl.BlockSpec((R, D0), lambda r: (r, 0)),
            pl.BlockSpec((NC, R, _DEG_W), lambda r: (0, r, 0)),
            pl.BlockSpec((D0, H1), lambda r: (0, 0)),
            pl.BlockSpec((H1, D0), lambda r: (0, 0)),
        ],
        out_specs=pl.BlockSpec((R, D0), lambda r: (r, 0)),
        out_shape=jax.ShapeDtypeStruct((N, D0), jnp.float32),
    )(s0p, g0, degp, W1, W2)


def _tc_final(s2p, g2, degp):
    def body(sp_ref, g_ref, dp_ref, o_ref):
        acc = sp_ref[0, :, 0:D2] + sp_ref[1, :, 0:D2] + g_ref[:, 0:D2]
        o_ref[...] = acc * _dinv_from(dp_ref)

    return pl.pallas_call(
        body,
        grid=(N // R,),
        in_specs=[
            pl.BlockSpec((NC, R, D0), lambda r: (0, r, 0)),
            pl.BlockSpec((R, D0), lambda r: (r, 0)),
            pl.BlockSpec((NC, R, _DEG_W), lambda r: (0, r, 0)),
        ],
        out_specs=pl.BlockSpec((R, D2), lambda r: (r, 0)),
        out_shape=jax.ShapeDtypeStruct((N, D2), jnp.float32),
    )(s2p, g2, degp)


def kernel(x, edge_index, W1, W2):
    src = edge_index[0].astype(jnp.int32).reshape(NW, C, K)
    dst = edge_index[1].astype(jnp.int32).reshape(NW, C, K)

    W2p = jnp.concatenate(
        [W2, jnp.zeros((H1, D0 - D2), jnp.float32)], axis=1)

    degp = _degree_partials(dst).reshape(NC, NP, _DEG_W)
    g0 = _tc_scale(x, degp)
    s0p = _propagate_partials(g0, src, dst, D0).reshape(NC, NP, D0)
    g2 = _tc_dense(s0p, g0, degp, W1, W2p)
    s2p = _propagate_partials(g2, src, dst, D0).reshape(NC, NP, D0)
    return _tc_final(s2p, g2, degp)
